# baseline (speedup 1.0000x reference)
import functools

import jax
import jax.numpy as jnp
from jax import lax
from jax.experimental import pallas as pl
from jax.experimental.pallas import tpu as pltpu
from jax.experimental.pallas import tpu_sc as plsc

_N = 10000
_E = 320000
_D = 128

_NC = 2
_NS = 16
_NW = _NC * _NS
_EPW = _E // _NW
_B = 80
_K = _EPW // _B
_WC = 5
_ROWS_PER_TILE = 640


def _sc_segment_sum(x, src_r, dst_r):
    mesh = plsc.VectorSubcoreMesh(core_axis_name="c", subcore_axis_name="s")

    @functools.partial(
        pl.kernel,
        out_type=jax.ShapeDtypeStruct((_NC, _N, _D), jnp.float32),
        mesh=mesh,
        scratch_types=[
            pltpu.VMEM((_WC, _B), jnp.int32),
            pltpu.VMEM((_WC, _B), jnp.int32),
            pltpu.VMEM((_WC, _B), jnp.int32),
            pltpu.VMEM((_WC, _B), jnp.int32),
            pltpu.VMEM((_B, _D), jnp.float32),
            pltpu.VMEM((_B, _D), jnp.float32),
            pltpu.VMEM((_B, _D), jnp.float32),
            pltpu.VMEM_SHARED((_N, _D), jnp.float32),
            pltpu.SemaphoreType.DMA,
            pltpu.SemaphoreType.DMA,
            pltpu.SemaphoreType.DMA,
            pltpu.SemaphoreType.DMA,
        ],
    )
    def k(x_hbm, src_hbm, dst_hbm, out_hbm, src_w0, src_w1, dst_w0, dst_w1,
          rows0, rows1, rows2, acc_sh, sem0, sem1, sem2, semw):
        c = lax.axis_index("c")
        s = lax.axis_index("s")
        wid = s * _NC + c

        zbase = jnp.minimum(s * _ROWS_PER_TILE, _N - _ROWS_PER_TILE)

        @pl.when(c == 0)
        def _():
            pltpu.sync_copy(
                x_hbm.at[pl.ds(zbase, _ROWS_PER_TILE)],
                acc_sh.at[pl.ds(zbase, _ROWS_PER_TILE)],
            )

        @pl.when(c == 1)
        def _():
            def zero_body(i, _):
                zr = i // (_D // 16)
                zc = (i % (_D // 16)) * 16
                rows0[zr, pl.ds(zc, 16)] = jnp.zeros((16,), jnp.float32)
                return 0

            lax.fori_loop(0, _B * (_D // 16), zero_body, 0)
            for rblk in range(_ROWS_PER_TILE // _B):
                pltpu.sync_copy(rows0, acc_sh.at[pl.ds(zbase + rblk * _B, _B)])

        pltpu.sync_copy(src_hbm.at[wid, 0], src_w0)
        pltpu.sync_copy(dst_hbm.at[wid, 0], dst_w0)
        pltpu.async_copy(src_hbm.at[wid, 1], src_w1, semw)
        pltpu.async_copy(dst_hbm.at[wid, 1], dst_w1, semw)

        plsc.subcore_barrier()

        def gather(x, buf, sem):
            r = x % _WC

            @pl.when((x // _WC) % 2 == 0)
            def _():
                pltpu.async_copy(x_hbm.at[src_w0.at[r]], buf, sem)

            @pl.when((x // _WC) % 2 == 1)
            def _():
                pltpu.async_copy(x_hbm.at[src_w1.at[r]], buf, sem)

        def scatter(j, buf):
            r = j % _WC

            @pl.when((j // _WC) % 2 == 0)
            def _():
                pltpu.sync_copy(buf, acc_sh.at[dst_w0.at[r]], add=True)

            @pl.when((j // _WC) % 2 == 1)
            def _():
                pltpu.sync_copy(buf, acc_sh.at[dst_w1.at[r]], add=True)

        def wait_gather(buf, sem):
            pltpu.make_async_copy(x_hbm.at[src_w0.at[0]], buf, sem).wait()

        gather(0, rows0, sem0)
        gather(1, rows1, sem1)

        def step(cur, csem, nxt, nsem, j):
            @pl.when(jnp.logical_and(j % _WC == 0, j > 0))
            def _():
                wnext = j // _WC + 1

                @pl.when(wnext < _K // _WC)
                def _():
                    @pl.when(wnext % 2 == 0)
                    def _():
                        pltpu.async_copy(src_hbm.at[wid, wnext], src_w0, semw)
                        pltpu.async_copy(dst_hbm.at[wid, wnext], dst_w0, semw)

                    @pl.when(wnext % 2 == 1)
                    def _():
                        pltpu.async_copy(src_hbm.at[wid, wnext], src_w1, semw)
                        pltpu.async_copy(dst_hbm.at[wid, wnext], dst_w1, semw)

            @pl.when(j % _WC == _WC - 2)
            def _():
                pltpu.make_async_copy(src_hbm.at[wid, 0], src_w0, semw).wait()
                pltpu.make_async_copy(dst_hbm.at[wid, 0], dst_w0, semw).wait()

            gather(j + 2, nxt, nsem)
            wait_gather(cur, csem)
            scatter(j, cur)

        def body(j, _):
            @pl.when(j % 3 == 0)
            def _():
                step(rows0, sem0, rows2, sem2, j)

            @pl.when(j % 3 == 1)
            def _():
                step(rows1, sem1, rows0, sem0, j)

            @pl.when(j % 3 == 2)
            def _():
                step(rows2, sem2, rows1, sem1, j)

            return 0

        lax.fori_loop(0, _K - 2, body, 0)
        wait_gather(rows0, sem0)
        scatter(_K - 2, rows0)
        wait_gather(rows1, sem1)
        scatter(_K - 1, rows1)

        plsc.subcore_barrier()

        @pl.when(s < _NS - 1)
        def _():
            pltpu.sync_copy(
                acc_sh.at[pl.ds(s * _ROWS_PER_TILE, _ROWS_PER_TILE)],
                out_hbm.at[c, pl.ds(s * _ROWS_PER_TILE, _ROWS_PER_TILE)],
            )

        @pl.when(s == _NS - 1)
        def _():
            last = (_NS - 1) * _ROWS_PER_TILE
            pltpu.sync_copy(
                acc_sh.at[pl.ds(last, _N - last)],
                out_hbm.at[c, pl.ds(last, _N - last)],
            )

    return k(x, src_r, dst_r)


_TR = 2000


def _tc_linear(p, w, b, lsm):
    def body(p_ref, w_ref, b_ref, o_ref):
        rst = p_ref[0] + p_ref[1]
        h = jnp.dot(rst, w_ref[...], preferred_element_type=jnp.float32) + b_ref[...]
        if lsm:
            m = jnp.max(h, axis=-1, keepdims=True)
            e = jnp.exp(h - m)
            h = h - m - jnp.log(jnp.sum(e, axis=-1, keepdims=True))
        o_ref[...] = h

    return pl.pallas_call(
        body,
        grid=(_N // _TR,),
        in_specs=[
            pl.BlockSpec((_NC, _TR, _D), lambda i: (0, i, 0)),
            pl.BlockSpec((_D, _D), lambda i: (0, 0)),
            pl.BlockSpec((1, _D), lambda i: (0, 0)),
        ],
        out_specs=pl.BlockSpec((_TR, _D), lambda i: (i, 0)),
        out_shape=jax.ShapeDtypeStruct((_N, _D), jnp.float32),
    )(p, w, b)


def kernel(features, edge_index, W1, b1, W2, b2):
    src_r = edge_index[0].reshape(_NW, _K // _WC, _WC, _B)
    dst_r = edge_index[1].reshape(_NW, _K // _WC, _WC, _B)
    b1r = b1.reshape(1, _D)
    b2r = b2.reshape(1, _D)

    p1 = _sc_segment_sum(features, src_r, dst_r)
    h1 = _tc_linear(p1, W1, b1r, lsm=False)
    p2 = _sc_segment_sum(h1, src_r, dst_r)
    return _tc_linear(p2, W2, b2r, lsm=True)


# speedup vs baseline: 1.0448x; 1.0448x over previous
import functools

import jax
import jax.numpy as jnp
from jax import lax
from jax.experimental import pallas as pl
from jax.experimental.pallas import tpu as pltpu
from jax.experimental.pallas import tpu_sc as plsc

_N = 10000
_E = 320000
_D = 128

_NC = 2
_NS = 16
_NW = _NC * _NS
_EPW = _E // _NW
_B = 80
_K = _EPW // _B
_WC = 5
_ROWS_PER_TILE = 640


def _sc_segment_sum(x, ei_r):
    mesh = plsc.VectorSubcoreMesh(core_axis_name="c", subcore_axis_name="s")

    @functools.partial(
        pl.kernel,
        out_type=jax.ShapeDtypeStruct((_NC, _N, _D), jnp.float32),
        mesh=mesh,
        scratch_types=[
            pltpu.VMEM((_WC, _B), jnp.int32),
            pltpu.VMEM((_WC, _B), jnp.int32),
            pltpu.VMEM((_WC, _B), jnp.int32),
            pltpu.VMEM((_WC, _B), jnp.int32),
            pltpu.VMEM((_B, _D), jnp.float32),
            pltpu.VMEM((_B, _D), jnp.float32),
            pltpu.VMEM((_B, _D), jnp.float32),
            pltpu.VMEM_SHARED((_N, _D), jnp.float32),
            pltpu.SemaphoreType.DMA,
            pltpu.SemaphoreType.DMA,
            pltpu.SemaphoreType.DMA,
            pltpu.SemaphoreType.DMA,
        ],
    )
    def k(x_hbm, ei_hbm, out_hbm, src_w0, src_w1, dst_w0, dst_w1,
          rows0, rows1, rows2, acc_sh, sem0, sem1, sem2, semw):
        c = lax.axis_index("c")
        s = lax.axis_index("s")
        wid = s * _NC + c

        zbase = jnp.minimum(s * _ROWS_PER_TILE, _N - _ROWS_PER_TILE)

        @pl.when(c == 0)
        def _():
            pltpu.sync_copy(
                x_hbm.at[pl.ds(zbase, _ROWS_PER_TILE)],
                acc_sh.at[pl.ds(zbase, _ROWS_PER_TILE)],
            )

        @pl.when(c == 1)
        def _():
            def zero_body(i, _):
                zr = i // (_D // 16)
                zc = (i % (_D // 16)) * 16
                rows0[zr, pl.ds(zc, 16)] = jnp.zeros((16,), jnp.float32)
                return 0

            lax.fori_loop(0, _B * (_D // 16), zero_body, 0)
            for rblk in range(_ROWS_PER_TILE // _B):
                pltpu.sync_copy(rows0, acc_sh.at[pl.ds(zbase + rblk * _B, _B)])

        pltpu.sync_copy(ei_hbm.at[0, wid, 0], src_w0)
        pltpu.sync_copy(ei_hbm.at[1, wid, 0], dst_w0)
        pltpu.async_copy(ei_hbm.at[0, wid, 1], src_w1, semw)
        pltpu.async_copy(ei_hbm.at[1, wid, 1], dst_w1, semw)

        plsc.subcore_barrier()

        def gather(x, buf, sem):
            r = x % _WC

            @pl.when((x // _WC) % 2 == 0)
            def _():
                pltpu.async_copy(x_hbm.at[src_w0.at[r]], buf, sem)

            @pl.when((x // _WC) % 2 == 1)
            def _():
                pltpu.async_copy(x_hbm.at[src_w1.at[r]], buf, sem)

        def scatter(j, buf):
            r = j % _WC

            @pl.when((j // _WC) % 2 == 0)
            def _():
                pltpu.sync_copy(buf, acc_sh.at[dst_w0.at[r]], add=True)

            @pl.when((j // _WC) % 2 == 1)
            def _():
                pltpu.sync_copy(buf, acc_sh.at[dst_w1.at[r]], add=True)

        def wait_gather(buf, sem):
            pltpu.make_async_copy(x_hbm.at[src_w0.at[0]], buf, sem).wait()

        gather(0, rows0, sem0)
        gather(1, rows1, sem1)

        def step(cur, csem, nxt, nsem, j):
            @pl.when(jnp.logical_and(j % _WC == 0, j > 0))
            def _():
                wnext = j // _WC + 1

                @pl.when(wnext < _K // _WC)
                def _():
                    @pl.when(wnext % 2 == 0)
                    def _():
                        pltpu.async_copy(ei_hbm.at[0, wid, wnext], src_w0, semw)
                        pltpu.async_copy(ei_hbm.at[1, wid, wnext], dst_w0, semw)

                    @pl.when(wnext % 2 == 1)
                    def _():
                        pltpu.async_copy(ei_hbm.at[0, wid, wnext], src_w1, semw)
                        pltpu.async_copy(ei_hbm.at[1, wid, wnext], dst_w1, semw)

            @pl.when(j % _WC == _WC - 2)
            def _():
                pltpu.make_async_copy(ei_hbm.at[0, wid, 0], src_w0, semw).wait()
                pltpu.make_async_copy(ei_hbm.at[1, wid, 0], dst_w0, semw).wait()

            gather(j + 2, nxt, nsem)
            wait_gather(cur, csem)
            scatter(j, cur)

        def body(j, _):
            @pl.when(j % 3 == 0)
            def _():
                step(rows0, sem0, rows2, sem2, j)

            @pl.when(j % 3 == 1)
            def _():
                step(rows1, sem1, rows0, sem0, j)

            @pl.when(j % 3 == 2)
            def _():
                step(rows2, sem2, rows1, sem1, j)

            return 0

        lax.fori_loop(0, _K - 2, body, 0)
        wait_gather(rows0, sem0)
        scatter(_K - 2, rows0)
        wait_gather(rows1, sem1)
        scatter(_K - 1, rows1)

        plsc.subcore_barrier()

        @pl.when(s < _NS - 1)
        def _():
            pltpu.sync_copy(
                acc_sh.at[pl.ds(s * _ROWS_PER_TILE, _ROWS_PER_TILE)],
                out_hbm.at[c, pl.ds(s * _ROWS_PER_TILE, _ROWS_PER_TILE)],
            )

        @pl.when(s == _NS - 1)
        def _():
            last = (_NS - 1) * _ROWS_PER_TILE
            pltpu.sync_copy(
                acc_sh.at[pl.ds(last, _N - last)],
                out_hbm.at[c, pl.ds(last, _N - last)],
            )

    return k(x, ei_r)


_TR = 2000


def _tc_linear(p, w, b, lsm):
    def body(p_ref, w_ref, b_ref, o_ref):
        rst = p_ref[0] + p_ref[1]
        h = jnp.dot(rst, w_ref[...], preferred_element_type=jnp.float32) + b_ref[...]
        if lsm:
            m = jnp.max(h, axis=-1, keepdims=True)
            e = jnp.exp(h - m)
            h = h - m - jnp.log(jnp.sum(e, axis=-1, keepdims=True))
        o_ref[...] = h

    return pl.pallas_call(
        body,
        grid=(_N // _TR,),
        in_specs=[
            pl.BlockSpec((_NC, _TR, _D), lambda i: (0, i, 0)),
            pl.BlockSpec((_D, _D), lambda i: (0, 0)),
            pl.BlockSpec((1, _D), lambda i: (0, 0)),
        ],
        out_specs=pl.BlockSpec((_TR, _D), lambda i: (i, 0)),
        out_shape=jax.ShapeDtypeStruct((_N, _D), jnp.float32),
    )(p, w, b)


def kernel(features, edge_index, W1, b1, W2, b2):
    ei_r = edge_index.reshape(2, _NW, _K // _WC, _WC, _B)
    b1r = b1.reshape(1, _D)
    b2r = b2.reshape(1, _D)

    p1 = _sc_segment_sum(features, ei_r)
    h1 = _tc_linear(p1, W1, b1r, lsm=False)
    p2 = _sc_segment_sum(h1, ei_r)
    return _tc_linear(p2, W2, b2r, lsm=True)
